# Initial kernel scaffold; baseline (speedup 1.0000x reference)
import jax
import jax.numpy as jnp
from jax.experimental import pallas as pl


def kernel(x, edge_index, edge_type, W1, b1, W2, b2):
    raise NotImplementedError("write your pallas kernel here")


# speedup vs baseline: 8.7454x; 8.7454x over previous
import functools

import jax
import jax.numpy as jnp
from jax import lax
from jax.experimental import pallas as pl
from jax.experimental.pallas import tpu as pltpu
from jax.experimental.pallas import tpu_sc as plsc

N = 10000
E = 320000
R = 3
D = 128
DH = 64

NSUB = 16
EPT = E // NSUB
BLK = 400
NBLK = EPT // BLK
CH = 80
NCH = BLK // CH

FLUSH = CH
NFLUSH = (R * N) // FLUSH
FL_PT = -(-NFLUSH // NSUB)
DEGP = 30720
DEG_PT = DEGP // NSUB


def _sc_body(with_deg, *refs):
    if with_deg:
        (table, src_h, dst_h, et_h, zrow_h, zdeg_h, s_out, deg_out,
         src_v, dst_v, et_v, gidx_v, sidx_v, rows_v, ones_v,
         zdeg_v, acc_sh, deg_sh) = refs
    else:
        (table, src_h, dst_h, et_h, zrow_h, s_out,
         src_v, dst_v, et_v, gidx_v, sidx_v, rows_v,
         acc_sh) = refs

    c = lax.axis_index("c")
    s = lax.axis_index("s")
    cN = c * N

    pltpu.sync_copy(zrow_h, rows_v)
    for t in range(FL_PT):
        idx = t * NSUB + s

        @pl.when(idx < NFLUSH)
        def _():
            pltpu.sync_copy(rows_v, acc_sh.at[pl.ds(idx * FLUSH, FLUSH)])
    if with_deg:
        pltpu.sync_copy(zdeg_h, zdeg_v)
        pltpu.sync_copy(zdeg_v, deg_sh.at[pl.ds(s * DEG_PT, DEG_PT)])
        for k in range(CH // 16):
            ones_v[pl.ds(k * 16, 16)] = jnp.ones((16,), jnp.float32)
    plsc.subcore_barrier()

    def block_body(b, _):
        base = s * EPT + b * BLK
        pltpu.sync_copy(src_h.at[pl.ds(base, BLK)], src_v)
        pltpu.sync_copy(dst_h.at[pl.ds(base, BLK)], dst_v)
        pltpu.sync_copy(et_h.at[pl.ds(base, BLK)], et_v)

        def chunk_body(ch, _):
            for k in range(CH // 16):
                o = ch * CH + k * 16
                s16 = src_v[pl.ds(o, 16)]
                gidx_v[0, pl.ds(k * 16, 16)] = s16 + cN
                si16 = et_v[pl.ds(o, 16)] * N + dst_v[pl.ds(o, 16)]
                sidx_v[0, pl.ds(k * 16, 16)] = si16
            pltpu.sync_copy(table.at[gidx_v.at[0]], rows_v)
            pltpu.sync_copy(rows_v, acc_sh.at[sidx_v.at[0]], add=True)
            if with_deg:
                pltpu.sync_copy(ones_v, deg_sh.at[sidx_v.at[0]], add=True)
            return 0

        lax.fori_loop(0, NCH, chunk_body, 0)
        return 0

    lax.fori_loop(0, NBLK, block_body, 0)
    plsc.subcore_barrier()

    for t in range(FL_PT):
        idx = t * NSUB + s

        @pl.when(idx < NFLUSH)
        def _():
            off = idx * FLUSH
            pltpu.sync_copy(acc_sh.at[pl.ds(off, FLUSH)], rows_v)
            pltpu.sync_copy(rows_v, s_out.at[c, pl.ds(off, FLUSH)])
    if with_deg:
        @pl.when(c == 0)
        def _():
            pltpu.sync_copy(deg_sh.at[pl.ds(s * DEG_PT, DEG_PT)], zdeg_v)
            pltpu.sync_copy(zdeg_v, deg_out.at[pl.ds(s * DEG_PT, DEG_PT)])


def _make_sc_agg(with_deg):
    out_type = [jax.ShapeDtypeStruct((2, R * N, DH), jnp.float32)]
    scratch = [
        pltpu.VMEM((BLK,), jnp.int32),
        pltpu.VMEM((BLK,), jnp.int32),
        pltpu.VMEM((BLK,), jnp.int32),
        pltpu.VMEM((1, CH), jnp.int32),
        pltpu.VMEM((1, CH), jnp.int32),
        pltpu.VMEM((CH, DH), jnp.float32),
    ]
    if with_deg:
        out_type.append(jax.ShapeDtypeStruct((DEGP,), jnp.float32))
        scratch.append(pltpu.VMEM((CH,), jnp.float32))
        scratch.append(pltpu.VMEM((DEG_PT,), jnp.float32))
    scratch.append(pltpu.VMEM_SHARED((R * N, DH), jnp.float32))
    if with_deg:
        scratch.append(pltpu.VMEM_SHARED((DEGP,), jnp.float32))

    mesh = plsc.VectorSubcoreMesh(core_axis_name="c", subcore_axis_name="s")
    return pl.kernel(
        functools.partial(_sc_body, with_deg),
        out_type=out_type,
        mesh=mesh,
        scratch_types=scratch,
        compiler_params=pltpu.CompilerParams(use_tc_tiling_on_sc=False),
    )


def _tc_body(split_out, nrows, s_ref, deg_ref, w_ref, b_ref, out_ref):
    bsum = b_ref[0] + b_ref[1] + b_ref[2]
    acc = jnp.broadcast_to(bsum[None, :], (nrows, D)).astype(jnp.float32)
    for r in range(R):
        sr = jnp.concatenate([s_ref[0, r], s_ref[1, r]], axis=-1)
        nr = 1.0 / jnp.maximum(deg_ref[0, r], 1.0)
        acc = acc + jnp.dot(sr * nr[:, None], w_ref[r],
                            preferred_element_type=jnp.float32)
    if split_out:
        out_ref[0] = acc[:, :DH]
        out_ref[1] = acc[:, DH:]
    else:
        out_ref[...] = acc


def _tc_layer(s4, deg3, w, b, split_out):
    nblocks = 10
    nrows = N // nblocks
    in_specs = [
        pl.BlockSpec((2, R, nrows, DH), lambda i: (0, 0, i, 0)),
        pl.BlockSpec((1, R, nrows), lambda i: (i, 0, 0)),
        pl.BlockSpec((R, D, D), lambda i: (0, 0, 0)),
        pl.BlockSpec((R, D), lambda i: (0, 0)),
    ]
    if split_out:
        out_shape = jax.ShapeDtypeStruct((2, N, DH), jnp.float32)
        out_spec = pl.BlockSpec((2, nrows, DH), lambda i: (0, i, 0))
    else:
        out_shape = jax.ShapeDtypeStruct((N, D), jnp.float32)
        out_spec = pl.BlockSpec((nrows, D), lambda i: (i, 0))
    return pl.pallas_call(
        functools.partial(_tc_body, split_out, nrows),
        grid=(nblocks,),
        in_specs=in_specs,
        out_specs=out_spec,
        out_shape=out_shape,
    )(s4, deg3, w, b)


def kernel(x, edge_index, edge_type, W1, b1, W2, b2):
    src = edge_index[0]
    dst = edge_index[1]
    et = edge_type

    x_split = jnp.concatenate([x[:, :DH], x[:, DH:]], axis=0)
    zrow = jnp.zeros((CH, DH), jnp.float32)
    zdeg = jnp.zeros((DEG_PT,), jnp.float32)

    s1, deg = _make_sc_agg(True)(x_split, src, dst, et, zrow, zdeg)
    deg3 = deg[: R * N].reshape(R, 10, N // 10).transpose(1, 0, 2)
    h_split = _tc_layer(s1.reshape(2, R, N, DH), deg3, W1, b1, True)
    s2, = _make_sc_agg(False)(h_split.reshape(2 * N, DH), src, dst, et, zrow)
    out = _tc_layer(s2.reshape(2, R, N, DH), deg3, W2, b2, False)
    return out
